# baseline (speedup 1.0000x reference)
import jax
import jax.numpy as jnp
from jax.experimental import pallas as pl
from jax.experimental.pallas import tpu as pltpu

_B, _N, _M, _D = 4, 4096, 16384, 3
_KX, _KS, _P, _H, _O = 256, 128, 128, 256, 128
_K = 16
_MB = 256
_W = 1280
_INF = jnp.inf


def _make_kernel(n, blocks_per_par):

  def body(path_ref, s1_ref, s2_ref, blof_ref,
           par_ref, posT_ref, bx_ref, x_ref,
           ps_ref, bs_ref, xs_ref,
           W1_ref, b1_ref, W2_ref, b2_ref, Wp_ref, bp_ref,
           out_ref,
           keys_ref, yacc_ref, wacc_ref):
    i = pl.program_id(0)
    ps = ps_ref[...]
    py2 = jnp.sum(ps * ps, axis=1, keepdims=True)

    def run_path(width, s, rowmask=None, accumulate=False):
      if s is None:
        csl = slice(None)
        rsl = slice(None)
      else:
        csl = pl.ds(s, width)
        rsl = pl.ds(s, width)
      posTw = posT_ref[:, csl]
      px2 = jnp.sum(posTw * posTw, axis=0, keepdims=True)
      d2 = py2 + px2 - 2.0 * jnp.dot(ps, posTw,
                                     preferred_element_type=jnp.float32)
      d2 = jnp.where(bs_ref[...] != bx_ref[:, csl], _INF, d2)
      keys_ref[:, :width] = d2

      t = jnp.full((_MB, 1), -_INF, jnp.float32)
      for _ in range(_K):
        c = keys_ref[:, :width]
        t = jnp.min(jnp.where(c > t, c, _INF), axis=1, keepdims=True)

      d2v = keys_ref[:, :width]
      w = jnp.where(d2v <= t, 1.0 / jnp.maximum(d2v, 1e-16), 0.0)
      if rowmask is not None:
        w = w * rowmask
      ws = jnp.sum(w, axis=1, keepdims=True)
      yv = jnp.dot(w, x_ref[rsl, :], preferred_element_type=jnp.float32)
      if accumulate:
        wacc_ref[...] += ws
        yacc_ref[...] += yv
      else:
        wacc_ref[...] = ws
        yacc_ref[...] = yv

    @pl.when(path_ref[i] == 1)
    def _():
      run_path(_W, pl.multiple_of(s1_ref[i], 128))

    @pl.when(path_ref[i] == 2)
    def _():
      mask_lo = jnp.where(bs_ref[...] == blof_ref[i], 1.0, 0.0)
      run_path(_W, pl.multiple_of(s1_ref[i], 128), rowmask=mask_lo)
      run_path(_W, pl.multiple_of(s2_ref[i], 128), rowmask=1.0 - mask_lo,
               accumulate=True)

    @pl.when(path_ref[i] == 0)
    def _():
      run_path(n, None)

    y = yacc_ref[...] / wacc_ref[...]
    xc = jnp.concatenate([y, xs_ref[...]], axis=1)
    h = jnp.tanh(jnp.dot(xc, W1_ref[...], preferred_element_type=jnp.float32)
                 + b1_ref[...])
    h = jnp.dot(h, W2_ref[...], preferred_element_type=jnp.float32) + b2_ref[...]
    g_all = jnp.tanh(jnp.dot(par_ref[...], Wp_ref[...],
                             preferred_element_type=jnp.float32)
                     + bp_ref[...])
    pid = pl.program_id(0) // blocks_per_par
    rows = jax.lax.broadcasted_iota(jnp.int32, g_all.shape, 0)
    g = jnp.sum(jnp.where(rows == pid, g_all, 0.0), axis=0, keepdims=True)
    out_ref[...] = h * g

  return body


def kernel(par_embedding, x, pos, batch, x_skip, pos_skip, batch_skip,
           W1, b1, W2, b2, Wp, bp):
    M, N = pos_skip.shape[0], pos.shape[0]
    n_repeats = M // par_embedding.shape[0]
    par_rows = par_embedding.reshape(par_embedding.shape[0], par_embedding.shape[-1])
    posT = pos.T
    batch = batch.astype(jnp.int32)
    batch_skip = batch_skip.astype(jnp.int32)
    bx = batch.astype(jnp.float32).reshape(1, N)
    bs = batch_skip.astype(jnp.float32).reshape(M, 1)

    nblocks = M // _MB
    blk_lo = batch_skip[:: _MB]
    blk_hi = batch_skip[_MB - 1:: _MB]
    lo_l = jnp.searchsorted(batch, blk_lo, side="left").astype(jnp.int32)
    lo_h = (jnp.searchsorted(batch, blk_lo, side="right") - 1).astype(jnp.int32)
    hi_l = jnp.searchsorted(batch, blk_hi, side="left").astype(jnp.int32)
    hi_h = (jnp.searchsorted(batch, blk_hi, side="right") - 1).astype(jnp.int32)
    a1 = (lo_l // 128) * 128
    a2 = (hi_l // 128) * 128
    narrow = (hi_h - a1 + 1) <= _W
    dual = (jnp.logical_not(narrow)
            & (blk_hi - blk_lo == 1)
            & ((lo_h - a1 + 1) <= _W)
            & ((hi_h - a2 + 1) <= _W))
    path = jnp.where(narrow, 1, jnp.where(dual, 2, 0)).astype(jnp.int32)
    s1 = jnp.minimum(a1, N - _W).astype(jnp.int32)
    s2 = jnp.minimum(a2, N - _W).astype(jnp.int32)
    blof = blk_lo.astype(jnp.float32)

    grid = (nblocks,)
    const = lambda i: (0, 0)
    smem = lambda shape: pl.BlockSpec(shape, lambda i: tuple(0 for _ in shape),
                                      memory_space=pltpu.SMEM)
    out = pl.pallas_call(
        _make_kernel(N, n_repeats // _MB),
        grid=grid,
        in_specs=[
            smem((nblocks,)), smem((nblocks,)), smem((nblocks,)), smem((nblocks,)),
            pl.BlockSpec((par_rows.shape[0], _P), const),
            pl.BlockSpec((_D, N), const),
            pl.BlockSpec((1, N), const),
            pl.BlockSpec((N, _KX), const),
            pl.BlockSpec((_MB, _D), lambda i: (i, 0)),
            pl.BlockSpec((_MB, 1), lambda i: (i, 0)),
            pl.BlockSpec((_MB, _KS), lambda i: (i, 0)),
            pl.BlockSpec((_KX + _KS, _H), const),
            pl.BlockSpec((1, _H), const),
            pl.BlockSpec((_H, _O), const),
            pl.BlockSpec((1, _O), const),
            pl.BlockSpec((_P, _O), const),
            pl.BlockSpec((1, _O), const),
        ],
        out_specs=pl.BlockSpec((_MB, _O), lambda i: (i, 0)),
        out_shape=jax.ShapeDtypeStruct((M, _O), jnp.float32),
        scratch_shapes=[
            pltpu.VMEM((_MB, N), jnp.float32),
            pltpu.VMEM((_MB, _KX), jnp.float32),
            pltpu.VMEM((_MB, 1), jnp.float32),
        ],
    )(path, s1, s2, blof,
      par_rows, posT, bx, x,
      pos_skip, bs, x_skip,
      W1, b1.reshape(1, _H), W2, b2.reshape(1, _O), Wp, bp.reshape(1, _O))
    return (out, pos_skip, batch_skip)


# speedup vs baseline: 1.0729x; 1.0729x over previous
import jax
import jax.numpy as jnp
from jax.experimental import pallas as pl
from jax.experimental.pallas import tpu as pltpu

_B, _N, _M, _D = 4, 4096, 16384, 3
_KX, _KS, _P, _H, _O = 256, 128, 128, 256, 128
_K = 16
_MB = 256
_W = 1280
_INF = jnp.inf


def _make_kernel(n, blocks_per_par):

  def body(start_ref, narrow_ref,
           par_ref, posT_ref, bx_ref, x_ref,
           ps_ref, bs_ref, xs_ref,
           W1_ref, b1_ref, W2_ref, b2_ref, Wp_ref, bp_ref,
           out_ref,
           keys_ref, yacc_ref, wacc_ref):
    i = pl.program_id(0)
    ps = ps_ref[...]
    py2 = jnp.sum(ps * ps, axis=1, keepdims=True)

    def run_path(width, s):
      if s is None:
        csl = slice(None)
        rsl = slice(None)
      else:
        csl = pl.ds(s, width)
        rsl = pl.ds(s, width)
      posTw = posT_ref[:, csl]
      px2 = jnp.sum(posTw * posTw, axis=0, keepdims=True)
      d2 = py2 + px2 - 2.0 * jnp.dot(ps, posTw,
                                     preferred_element_type=jnp.float32)
      d2 = jnp.where(bs_ref[...] != bx_ref[:, csl], _INF, d2)
      keys_ref[:, :width] = d2

      t = jnp.full((_MB, 1), -_INF, jnp.float32)
      for _ in range(_K):
        c = keys_ref[:, :width]
        t = jnp.min(jnp.where(c > t, c, _INF), axis=1, keepdims=True)

      d2v = keys_ref[:, :width]
      w = jnp.where(d2v <= t, 1.0 / jnp.maximum(d2v, 1e-16), 0.0)
      wacc_ref[...] = jnp.sum(w, axis=1, keepdims=True)
      yacc_ref[...] = jnp.dot(w, x_ref[rsl, :],
                              preferred_element_type=jnp.float32)

    @pl.when(narrow_ref[i] == 1)
    def _():
      run_path(_W, pl.multiple_of(start_ref[i], 128))

    @pl.when(narrow_ref[i] == 0)
    def _():
      run_path(n, None)

    y = yacc_ref[...] / wacc_ref[...]
    xc = jnp.concatenate([y, xs_ref[...]], axis=1)
    h = jnp.tanh(jnp.dot(xc, W1_ref[...], preferred_element_type=jnp.float32)
                 + b1_ref[...])
    h = jnp.dot(h, W2_ref[...], preferred_element_type=jnp.float32) + b2_ref[...]
    g_all = jnp.tanh(jnp.dot(par_ref[...], Wp_ref[...],
                             preferred_element_type=jnp.float32)
                     + bp_ref[...])
    pid = pl.program_id(0) // blocks_per_par
    rows = jax.lax.broadcasted_iota(jnp.int32, g_all.shape, 0)
    g = jnp.sum(jnp.where(rows == pid, g_all, 0.0), axis=0, keepdims=True)
    out_ref[...] = h * g

  return body


def kernel(par_embedding, x, pos, batch, x_skip, pos_skip, batch_skip,
           W1, b1, W2, b2, Wp, bp):
    M, N = pos_skip.shape[0], pos.shape[0]
    n_repeats = M // par_embedding.shape[0]
    par_rows = par_embedding.reshape(par_embedding.shape[0], par_embedding.shape[-1])
    posT = pos.T
    batch = batch.astype(jnp.int32)
    batch_skip = batch_skip.astype(jnp.int32)
    bx = batch.astype(jnp.float32).reshape(1, N)
    bs = batch_skip.astype(jnp.float32).reshape(M, 1)

    nblocks = M // _MB
    blk_lo = batch_skip[:: _MB]
    blk_hi = batch_skip[_MB - 1:: _MB]
    col_lo = jnp.searchsorted(batch, blk_lo, side="left").astype(jnp.int32)
    col_hi = (jnp.searchsorted(batch, blk_hi, side="right") - 1).astype(jnp.int32)
    a = (col_lo // 128) * 128
    narrow = ((col_hi - a + 1) <= _W).astype(jnp.int32)
    start = jnp.minimum(a, N - _W).astype(jnp.int32)

    grid = (nblocks,)
    const = lambda i: (0, 0)
    smem = lambda shape: pl.BlockSpec(shape, lambda i: tuple(0 for _ in shape),
                                      memory_space=pltpu.SMEM)
    out = pl.pallas_call(
        _make_kernel(N, n_repeats // _MB),
        grid=grid,
        in_specs=[
            smem((nblocks,)), smem((nblocks,)),
            pl.BlockSpec((par_rows.shape[0], _P), const),
            pl.BlockSpec((_D, N), const),
            pl.BlockSpec((1, N), const),
            pl.BlockSpec((N, _KX), const),
            pl.BlockSpec((_MB, _D), lambda i: (i, 0)),
            pl.BlockSpec((_MB, 1), lambda i: (i, 0)),
            pl.BlockSpec((_MB, _KS), lambda i: (i, 0)),
            pl.BlockSpec((_KX + _KS, _H), const),
            pl.BlockSpec((1, _H), const),
            pl.BlockSpec((_H, _O), const),
            pl.BlockSpec((1, _O), const),
            pl.BlockSpec((_P, _O), const),
            pl.BlockSpec((1, _O), const),
        ],
        out_specs=pl.BlockSpec((_MB, _O), lambda i: (i, 0)),
        out_shape=jax.ShapeDtypeStruct((M, _O), jnp.float32),
        scratch_shapes=[
            pltpu.VMEM((_MB, N), jnp.float32),
            pltpu.VMEM((_MB, _KX), jnp.float32),
            pltpu.VMEM((_MB, 1), jnp.float32),
        ],
    )(start, narrow,
      par_rows, posT, bx, x,
      pos_skip, bs, x_skip,
      W1, b1.reshape(1, _H), W2, b2.reshape(1, _O), Wp, bp.reshape(1, _O))
    return (out, pos_skip, batch_skip)
